# baseline (speedup 1.0000x reference)
import functools

import jax
import jax.numpy as jnp
from jax import lax
from jax.experimental import pallas as pl
from jax.experimental.pallas import tpu as pltpu
from jax.experimental.pallas import tpu_sc as plsc

_D = 768
_B = 32768
_EPS = 1e-5

_NC = 2
_NS = 16
_NW = _NC * _NS
_NSLICE = 8
_SB = _B // _NSLICE
_BPW = _SB // _NW
_CHUNK = 64
_NCHUNK = _BPW // _CHUNK


def _sc_gather_body(table_hbm, ids_hbm, out_hbm, idx_v, rows0, rows1, sem0, sem1):
    wid = lax.axis_index("s") * _NC + lax.axis_index("c")
    pltpu.sync_copy(ids_hbm.at[wid], idx_v)
    base = wid * _BPW
    bufs = (rows0, rows1)
    sems = (sem0, sem1)
    pltpu.async_copy(table_hbm.at[idx_v.at[0]], bufs[0], sems[0])
    for j in range(_NCHUNK):
        cur = j % 2
        if j + 1 < _NCHUNK:
            pltpu.async_copy(table_hbm.at[idx_v.at[j + 1]], bufs[1 - cur], sems[1 - cur])
        pltpu.make_async_copy(table_hbm.at[idx_v.at[j]], bufs[cur], sems[cur]).wait()
        pltpu.sync_copy(bufs[cur], out_hbm.at[pl.ds(base + j * _CHUNK, _CHUNK)])


_sc_gather = functools.partial(
    pl.kernel,
    out_type=jax.ShapeDtypeStruct((_SB, _D), jnp.float32),
    mesh=plsc.VectorSubcoreMesh(core_axis_name="c", subcore_axis_name="s"),
    scratch_types=[
        pltpu.VMEM((_NCHUNK, _CHUNK), jnp.int32),
        pltpu.VMEM((_CHUNK, _D), jnp.float32),
        pltpu.VMEM((_CHUNK, _D), jnp.float32),
        pltpu.SemaphoreType.DMA,
        pltpu.SemaphoreType.DMA,
    ],
)(_sc_gather_body)


_RBLK = 512
_NBLK = _SB // _RBLK


def _ln_block(rows_ref, rid_ref, role_ref, bias_ref, gamma_ref, beta_ref, out_ref):
    y = rows_ref[...]
    rid = rid_ref[...]
    rb = role_ref[...] + bias_ref[...]
    contrib = jnp.broadcast_to(rb[0:1, :], y.shape)
    for k in range(1, 4):
        contrib = jnp.where(rid == k, rb[k:k + 1, :], contrib)
    y = y + contrib
    mean = jnp.mean(y, axis=1, keepdims=True)
    yc = y - mean
    var = jnp.mean(yc * yc, axis=1, keepdims=True)
    normed = yc * lax.rsqrt(var + _EPS)
    out_ref[...] = normed * gamma_ref[...] + beta_ref[...]


def _tc_ln_first(rows_ref, rid_ref, role_ref, bias_ref, gamma_ref, beta_ref, out_ref):
    _ln_block(rows_ref, rid_ref, role_ref, bias_ref, gamma_ref, beta_ref, out_ref)


def _tc_ln_acc(acc_ref, rows_ref, rid_ref, role_ref, bias_ref, gamma_ref, beta_ref, out_ref):
    del acc_ref
    _ln_block(rows_ref, rid_ref, role_ref, bias_ref, gamma_ref, beta_ref, out_ref)


def _tc_ln_slice(s, acc, rows, rid2d, role_table, bias2d, gamma2d, beta2d):
    data_specs = [
        pl.BlockSpec((_RBLK, _D), lambda i: (i, 0)),
        pl.BlockSpec((_RBLK, 1), lambda i: (i, 0)),
        pl.BlockSpec((4, _D), lambda i: (0, 0)),
        pl.BlockSpec((1, _D), lambda i: (0, 0)),
        pl.BlockSpec((1, _D), lambda i: (0, 0)),
        pl.BlockSpec((1, _D), lambda i: (0, 0)),
    ]
    out_spec = pl.BlockSpec((_RBLK, _D), lambda i, s=s: (s * _NBLK + i, 0))
    common = dict(
        grid=(_NBLK,),
        out_specs=out_spec,
        out_shape=jax.ShapeDtypeStruct((_B, _D), jnp.float32),
    )
    args = (rows, rid2d, role_table, bias2d, gamma2d, beta2d)
    if s == 0:
        return pl.pallas_call(_tc_ln_first, in_specs=data_specs, **common)(*args)
    acc_spec = pl.BlockSpec((8, 128), lambda i: (0, 0))
    return pl.pallas_call(
        _tc_ln_acc,
        in_specs=[acc_spec] + data_specs,
        input_output_aliases={0: 0},
        **common,
    )(acc, *args)


def kernel(input_ids, role_ids, token_table, role_table, input_bias, ln_gamma, ln_beta):
    ids = input_ids.reshape(_NSLICE, _NW, _NCHUNK, _CHUNK).astype(jnp.int32)
    rids = role_ids.reshape(_NSLICE, _SB, 1).astype(jnp.int32)
    bias2d = input_bias.reshape(1, _D)
    gamma2d = ln_gamma.reshape(1, _D)
    beta2d = ln_beta.reshape(1, _D)
    acc = None
    for s in range(_NSLICE):
        gathered = _sc_gather(token_table, ids[s])
        acc = _tc_ln_slice(s, acc, gathered, rids[s], role_table, bias2d, gamma2d, beta2d)
    return acc.reshape(input_ids.shape[0], input_ids.shape[1], _D)


# speedup vs baseline: 1.0599x; 1.0599x over previous
import functools

import jax
import jax.numpy as jnp
from jax import lax
from jax.experimental import pallas as pl
from jax.experimental.pallas import tpu as pltpu
from jax.experimental.pallas import tpu_sc as plsc

_D = 768
_B = 32768
_EPS = 1e-5

_NC = 2
_NS = 16
_NW = _NC * _NS
_NSLICE = 4
_SB = _B // _NSLICE
_BPW = _SB // _NW
_CHUNK = 64
_NCHUNK = _BPW // _CHUNK


def _sc_gather_body(table_hbm, ids_hbm, out_hbm, idx_v, rows0, rows1, sem0, sem1):
    wid = lax.axis_index("s") * _NC + lax.axis_index("c")
    pltpu.sync_copy(ids_hbm.at[wid], idx_v)
    base = wid * _BPW
    bufs = (rows0, rows1)
    sems = (sem0, sem1)
    pltpu.async_copy(table_hbm.at[idx_v.at[0]], bufs[0], sems[0])
    for j in range(_NCHUNK):
        cur = j % 2
        if j + 1 < _NCHUNK:
            pltpu.async_copy(table_hbm.at[idx_v.at[j + 1]], bufs[1 - cur], sems[1 - cur])
        pltpu.make_async_copy(table_hbm.at[idx_v.at[j]], bufs[cur], sems[cur]).wait()
        pltpu.sync_copy(bufs[cur], out_hbm.at[pl.ds(base + j * _CHUNK, _CHUNK)])


_sc_gather = functools.partial(
    pl.kernel,
    out_type=jax.ShapeDtypeStruct((_SB, _D), jnp.float32),
    mesh=plsc.VectorSubcoreMesh(core_axis_name="c", subcore_axis_name="s"),
    scratch_types=[
        pltpu.VMEM((_NCHUNK, _CHUNK), jnp.int32),
        pltpu.VMEM((_CHUNK, _D), jnp.float32),
        pltpu.VMEM((_CHUNK, _D), jnp.float32),
        pltpu.SemaphoreType.DMA,
        pltpu.SemaphoreType.DMA,
    ],
)(_sc_gather_body)


_RBLK = 1024
_NBLK = _SB // _RBLK


def _ln_block(rows_ref, rid_ref, role_ref, bias_ref, gamma_ref, beta_ref, out_ref):
    y = rows_ref[...]
    rid = rid_ref[...]
    rb = role_ref[...] + bias_ref[...]
    contrib = jnp.broadcast_to(rb[0:1, :], y.shape)
    for k in range(1, 4):
        contrib = jnp.where(rid == k, rb[k:k + 1, :], contrib)
    y = y + contrib
    mean = jnp.mean(y, axis=1, keepdims=True)
    yc = y - mean
    var = jnp.mean(yc * yc, axis=1, keepdims=True)
    normed = yc * lax.rsqrt(var + _EPS)
    out_ref[...] = normed * gamma_ref[...] + beta_ref[...]


def _tc_ln_first(rows_ref, rid_ref, role_ref, bias_ref, gamma_ref, beta_ref, out_ref):
    _ln_block(rows_ref, rid_ref, role_ref, bias_ref, gamma_ref, beta_ref, out_ref)


def _tc_ln_acc(acc_ref, rows_ref, rid_ref, role_ref, bias_ref, gamma_ref, beta_ref, out_ref):
    del acc_ref
    _ln_block(rows_ref, rid_ref, role_ref, bias_ref, gamma_ref, beta_ref, out_ref)


def _tc_ln_slice(s, acc, rows, rid2d, role_table, bias2d, gamma2d, beta2d):
    data_specs = [
        pl.BlockSpec((_RBLK, _D), lambda i: (i, 0)),
        pl.BlockSpec((_RBLK, 1), lambda i: (i, 0)),
        pl.BlockSpec((4, _D), lambda i: (0, 0)),
        pl.BlockSpec((1, _D), lambda i: (0, 0)),
        pl.BlockSpec((1, _D), lambda i: (0, 0)),
        pl.BlockSpec((1, _D), lambda i: (0, 0)),
    ]
    out_spec = pl.BlockSpec((_RBLK, _D), lambda i, s=s: (s * _NBLK + i, 0))
    common = dict(
        grid=(_NBLK,),
        out_specs=out_spec,
        out_shape=jax.ShapeDtypeStruct((_B, _D), jnp.float32),
    )
    args = (rows, rid2d, role_table, bias2d, gamma2d, beta2d)
    if s == 0:
        return pl.pallas_call(_tc_ln_first, in_specs=data_specs, **common)(*args)
    acc_spec = pl.BlockSpec((8, 128), lambda i: (0, 0))
    return pl.pallas_call(
        _tc_ln_acc,
        in_specs=[acc_spec] + data_specs,
        input_output_aliases={0: 0},
        **common,
    )(acc, *args)


def kernel(input_ids, role_ids, token_table, role_table, input_bias, ln_gamma, ln_beta):
    ids = input_ids.reshape(_NSLICE, _NW, _NCHUNK, _CHUNK).astype(jnp.int32)
    rids = role_ids.reshape(_NSLICE, _SB, 1).astype(jnp.int32)
    bias2d = input_bias.reshape(1, _D)
    gamma2d = ln_gamma.reshape(1, _D)
    beta2d = ln_beta.reshape(1, _D)
    acc = None
    for s in range(_NSLICE):
        gathered = _sc_gather(token_table, ids[s])
        acc = _tc_ln_slice(s, acc, gathered, rids[s], role_table, bias2d, gamma2d, beta2d)
    return acc.reshape(input_ids.shape[0], input_ids.shape[1], _D)
